# baseline (speedup 1.0000x reference)
import functools

import jax
import jax.numpy as jnp
from jax import lax
from jax.experimental import pallas as pl
from jax.experimental.pallas import tpu as pltpu
from jax.experimental.pallas import tpu_sc as plsc

N = 10000
D = 256
E = 160000

NUM_TILES = 16
CHUNK = 80
EPAD = 161280
EDGES_PER_TILE = EPAD // NUM_TILES
CHUNKS_PER_TILE = EDGES_PER_TILE // CHUNK
PKW = 64
TW = 128
NPAD = 10240
ROWS_PER_TILE = NPAD // NUM_TILES
ZROWS = 80


def _sc_aggregate(pk0, pk1, src_r, dst_r):
  mesh = plsc.VectorSubcoreMesh(core_axis_name="c", subcore_axis_name="s")

  @functools.partial(
      pl.kernel,
      mesh=mesh,
      compiler_params=pltpu.CompilerParams(needs_layout_passes=False,
                                           use_tc_tiling_on_sc=False),
      out_type=[
          jax.ShapeDtypeStruct((NPAD, TW), jnp.float32),
          jax.ShapeDtypeStruct((NPAD, TW), jnp.float32),
          jax.ShapeDtypeStruct((2 * NUM_TILES, NPAD), jnp.float32),
      ],
      scratch_types=[
          pltpu.VMEM((CHUNK,), jnp.int32),
          pltpu.VMEM((CHUNK,), jnp.int32),
          pltpu.VMEM((CHUNK,), jnp.int32),
          pltpu.VMEM((CHUNK,), jnp.int32),
          pltpu.VMEM((CHUNK, PKW), jnp.int32),
          pltpu.VMEM((CHUNK, PKW), jnp.int32),
          pltpu.VMEM((CHUNK, TW), jnp.float32),
          pltpu.VMEM((CHUNK, TW), jnp.float32),
          pltpu.VMEM((NPAD,), jnp.float32),
          pltpu.VMEM_SHARED((NPAD, TW), jnp.float32),
          pltpu.SemaphoreType.DMA,
          pltpu.SemaphoreType.DMA,
          pltpu.SemaphoreType.DMA,
          pltpu.SemaphoreType.DMA,
          pltpu.SemaphoreType.DMA,
          pltpu.SemaphoreType.DMA,
      ],
  )
  def agg_kernel(pk0_hbm, pk1_hbm, src_hbm, dst_hbm,
                 out0_hbm, out1_hbm, parts_hbm,
                 src0_v, dst0_v, src1_v, dst1_v, pk0_v, pk1_v,
                 rows0_v, rows1_v,
                 deg_v, acc_sh, semg0, semg1, semi0, semi1, sems0, sems1):
    c = lax.axis_index("c")
    s = lax.axis_index("s")

    zvec = jnp.zeros((16,), jnp.float32)
    ones16 = jnp.ones((16,), jnp.float32)

    def zrow(r, _):
      for j in range(TW // 16):
        rows0_v[r, pl.ds(j * 16, 16)] = zvec
      return _

    lax.fori_loop(0, ZROWS, zrow, 0)

    def zdeg(r, _):
      deg_v[pl.ds(r * 16, 16)] = zvec
      return _

    lax.fori_loop(0, NPAD // 16, zdeg, 0)

    for z in range(ROWS_PER_TILE // ZROWS):
      pltpu.sync_copy(
          rows0_v.at[pl.ds(0, ZROWS)],
          acc_sh.at[pl.ds(s * ROWS_PER_TILE + z * ZROWS, ZROWS)])

    plsc.subcore_barrier()

    def run_core(t_hbm, out_hbm, parity, prow):
      bufs = ((src0_v, dst0_v, pk0_v, rows0_v, semg0, semi0, sems0),
              (src1_v, dst1_v, pk1_v, rows1_v, semg1, semi1, sems1))

      def fire_idx(j, b):
        srcb, dstb = bufs[b][0], bufs[b][1]
        semi = bufs[b][5]
        row = s * CHUNKS_PER_TILE + j
        pltpu.async_copy(src_hbm.at[row], srcb, semi)
        pltpu.async_copy(dst_hbm.at[row], dstb, semi)

      def wait_idx(b):
        srcb, dstb = bufs[b][0], bufs[b][1]
        semi = bufs[b][5]
        pltpu.make_async_copy(src_hbm.at[0], srcb, semi).wait()
        pltpu.make_async_copy(dst_hbm.at[0], dstb, semi).wait()

      def fire_gather(b):
        srcb, pkb, semg = bufs[b][0], bufs[b][2], bufs[b][4]
        pltpu.async_copy(t_hbm.at[srcb], pkb, semg)

      def wait_gather(b):
        srcb, pkb, semg = bufs[b][0], bufs[b][2], bufs[b][4]
        pltpu.make_async_copy(t_hbm.at[srcb], pkb, semg).wait()

      def convert(b):
        pkb, rows = bufs[b][2], bufs[b][3]

        def crow(r, _):
          for g in range(PKW // 16):
            w = pkb[r, pl.ds(g * 16, 16)]
            lo = plsc.bitcast(w << 16, jnp.float32)
            hi = plsc.bitcast(w & jnp.int32(-65536), jnp.float32)
            rows[r, pl.ds(g * 16, 16)] = lo
            rows[r, pl.ds(PKW + g * 16, 16)] = hi
          return _

        lax.fori_loop(0, CHUNK, crow, 0)

      def fire_scatter(b):
        dstb, rows, sems = bufs[b][1], bufs[b][3], bufs[b][6]
        pltpu.async_copy(rows, acc_sh.at[dstb], sems, add=True)

      def wait_scatter(b):
        dstb, rows, sems = bufs[b][1], bufs[b][3], bufs[b][6]
        pltpu.make_async_copy(rows, acc_sh.at[dstb], sems).wait()

      def count(b):
        if b == parity:
          dstb = bufs[b][1]
          for g in range(CHUNK // 16):
            dvec = dstb[pl.ds(g * 16, 16)]
            plsc.addupdate_scatter(deg_v, [dvec], ones16)

      fire_idx(0, 0)
      fire_idx(1, 1)
      wait_idx(0)
      fire_gather(0)

      def body(j2, _):
        j = 2 * j2
        wait_gather(0)
        convert(0)
        fire_scatter(0)
        count(0)
        wait_idx(1)
        fire_gather(1)
        wait_scatter(0)
        fire_idx(j + 2, 0)
        wait_gather(1)
        convert(1)
        fire_scatter(1)
        count(1)
        wait_idx(0)
        fire_gather(0)
        wait_scatter(1)
        fire_idx(j + 3, 1)
        return _

      lax.fori_loop(0, CHUNKS_PER_TILE // 2 - 1, body, 0)
      wait_gather(0)
      convert(0)
      fire_scatter(0)
      count(0)
      wait_idx(1)
      fire_gather(1)
      wait_scatter(0)
      wait_gather(1)
      convert(1)
      fire_scatter(1)
      count(1)
      wait_scatter(1)
      pltpu.sync_copy(deg_v, parts_hbm.at[prow])
      plsc.subcore_barrier()
      base = s * ROWS_PER_TILE
      pltpu.sync_copy(acc_sh.at[pl.ds(base, ROWS_PER_TILE)],
                      out_hbm.at[pl.ds(base, ROWS_PER_TILE)])

    @pl.when(c == 0)
    def _():
      run_core(pk0_hbm, out0_hbm, 0, s)

    @pl.when(c == 1)
    def _():
      run_core(pk1_hbm, out1_hbm, 1, NUM_TILES + s)

  return agg_kernel(pk0, pk1, src_r, dst_r)


def _layernorm(x, g, b, eps=1e-5):
  mu = jnp.mean(x, axis=-1, keepdims=True)
  var = jnp.mean((x - mu) ** 2, axis=-1, keepdims=True)
  return (x - mu) * jax.lax.rsqrt(var + eps) * g + b


ROW_BLK = 1000


def _dense_body(h_ref, a0_ref, a1_ref, deg_ref, A0_ref, A1_ref, bm_ref,
                WsT_ref, bs_ref, W1T_ref, bf1_ref, W2T_ref, bf2_ref,
                g1_ref, b1_ref, g2_ref, b2_ref, out_ref):
  h = h_ref[...]
  deg = jnp.sum(deg_ref[...], axis=1, keepdims=True)
  dot = functools.partial(jnp.dot, preferred_element_type=jnp.float32)
  pre = dot(a0_ref[...], A0_ref[...]) + dot(a1_ref[...], A1_ref[...])
  pre = pre + bm_ref[...] * deg
  agg = pre / jnp.maximum(deg, 1.0)
  x = h + jnp.maximum(dot(h, WsT_ref[...]) + bs_ref[...] + agg, 0.0)
  h1 = _layernorm(x, g1_ref[...], b1_ref[...])
  hid = dot(h1, W1T_ref[...]) + bf1_ref[...]
  hid = hid * 0.5 * (1.0 + lax.erf(hid * (2.0 ** -0.5)))
  ffn = dot(hid, W2T_ref[...]) + bf2_ref[...]
  out_ref[...] = _layernorm(h1 + ffn, g2_ref[...], b2_ref[...])


def _tc_dense(h, acc0, acc1, deg, A0, A1, bm, WsT, bs, W1T, bf1, W2T, bf2,
              g1, b1, g2, b2):
  grid = (N // ROW_BLK,)
  row_spec = lambda w: pl.BlockSpec((ROW_BLK, w), lambda i: (i, 0))
  full = lambda a: pl.BlockSpec(a.shape, lambda i: (0,) * a.ndim)
  return pl.pallas_call(
      _dense_body,
      grid=grid,
      in_specs=[
          row_spec(D), row_spec(TW), row_spec(TW), row_spec(2 * NUM_TILES),
          full(A0), full(A1), full(bm), full(WsT), full(bs),
          full(W1T), full(bf1), full(W2T), full(bf2),
          full(g1), full(b1), full(g2), full(b2),
      ],
      out_specs=row_spec(D),
      out_shape=jax.ShapeDtypeStruct((N, D), jnp.float32),
  )(h, acc0, acc1, deg, A0, A1, bm, WsT, bs, W1T, bf1, W2T, bf2,
    g1, b1, g2, b2)


@jax.jit
def kernel(h, edge_index, Wm, bm, Ws, bs, g1, beta1, W1, bf1, W2, bf2,
           g2, beta2):
  src = jnp.concatenate(
      [edge_index[0].astype(jnp.int32),
       jnp.zeros((EPAD - E,), jnp.int32)]).reshape(EPAD // CHUNK, CHUNK)
  dst = jnp.concatenate(
      [edge_index[1].astype(jnp.int32),
       N + jax.lax.rem(jnp.arange(EPAD - E, dtype=jnp.int32),
                       jnp.int32(NPAD - N))]).reshape(EPAD // CHUNK, CHUNK)

  hb = h.astype(jnp.bfloat16)
  packed = jax.lax.bitcast_convert_type(hb.reshape(N, D // 2, 2), jnp.int32)
  pk0 = packed[:, : PKW]
  pk1 = packed[:, PKW:]

  acc0, acc1, parts = _sc_aggregate(pk0, pk1, src, dst)
  deg = parts.T

  perm = jnp.concatenate(
      [jnp.arange(0, TW, 2, dtype=jnp.int32),
       jnp.arange(1, TW, 2, dtype=jnp.int32)])
  WmT = Wm.T
  A0 = WmT[: D // 2][perm]
  A1 = WmT[D // 2 :][perm]

  return _tc_dense(
      h, acc0, acc1, deg, A0, A1, bm[None, :], Ws.T, bs[None, :],
      W1.T, bf1[None, :], W2.T, bf2[None, :], g1[None, :], beta1[None, :],
      g2[None, :], beta2[None, :])


# speedup vs baseline: 1.8604x; 1.8604x over previous
import functools

import jax
import jax.numpy as jnp
from jax import lax
from jax.experimental import pallas as pl
from jax.experimental.pallas import tpu as pltpu
from jax.experimental.pallas import tpu_sc as plsc

N = 10000
D = 256
E = 160000

NUM_TILES = 16
CHUNK = 112
EPAD = 161280
EDGES_PER_TILE = EPAD // NUM_TILES
CHUNKS_PER_TILE = EDGES_PER_TILE // CHUNK
TW = 128
NPAD = 10240
ROWS_PER_TILE = NPAD // NUM_TILES
ZROWS = 80


def _sc_aggregate(h, src_r, dst_r):
  mesh = plsc.VectorSubcoreMesh(core_axis_name="c", subcore_axis_name="s")

  @functools.partial(
      pl.kernel,
      mesh=mesh,
      compiler_params=pltpu.CompilerParams(needs_layout_passes=False),
      out_type=[
          jax.ShapeDtypeStruct((NPAD, TW), jnp.float32),
          jax.ShapeDtypeStruct((NPAD, TW), jnp.float32),
          jax.ShapeDtypeStruct((2 * NUM_TILES, NPAD), jnp.float32),
      ],
      scratch_types=[
          pltpu.VMEM((CHUNK,), jnp.int32),
          pltpu.VMEM((CHUNK,), jnp.int32),
          pltpu.VMEM((CHUNK,), jnp.int32),
          pltpu.VMEM((CHUNK,), jnp.int32),
          pltpu.VMEM((CHUNK, TW), jnp.float32),
          pltpu.VMEM((CHUNK, TW), jnp.float32),
          pltpu.VMEM((NPAD,), jnp.float32),
          pltpu.VMEM_SHARED((NPAD, TW), jnp.float32),
          pltpu.SemaphoreType.DMA,
          pltpu.SemaphoreType.DMA,
          pltpu.SemaphoreType.DMA,
          pltpu.SemaphoreType.DMA,
          pltpu.SemaphoreType.DMA,
          pltpu.SemaphoreType.DMA,
      ],
  )
  def agg_kernel(h_hbm, src_hbm, dst_hbm,
                 out0_hbm, out1_hbm, parts_hbm,
                 src0_v, dst0_v, src1_v, dst1_v, rows0_v, rows1_v,
                 deg_v, acc_sh, semg0, semg1, semi0, semi1, sems0, sems1):
    c = lax.axis_index("c")
    s = lax.axis_index("s")

    zvec = jnp.zeros((16,), jnp.float32)
    ones16 = jnp.ones((16,), jnp.float32)

    def zrow(r, _):
      for j in range(TW // 16):
        rows0_v[r, pl.ds(j * 16, 16)] = zvec
      return _

    lax.fori_loop(0, ZROWS, zrow, 0)

    def zdeg(r, _):
      deg_v[pl.ds(r * 16, 16)] = zvec
      return _

    lax.fori_loop(0, NPAD // 16, zdeg, 0)

    for z in range(ROWS_PER_TILE // ZROWS):
      pltpu.sync_copy(
          rows0_v.at[pl.ds(0, ZROWS)],
          acc_sh.at[pl.ds(s * ROWS_PER_TILE + z * ZROWS, ZROWS)])

    plsc.subcore_barrier()

    def run_core(t_hbm, coff, out_hbm, parity, prow):
      bufs = ((src0_v, dst0_v, rows0_v, semg0, semi0, sems0),
              (src1_v, dst1_v, rows1_v, semg1, semi1, sems1))

      def fire_idx(j, b):
        srcb, dstb, _, _, semi, _ = bufs[b]
        row = s * CHUNKS_PER_TILE + j
        pltpu.async_copy(src_hbm.at[row], srcb, semi)
        pltpu.async_copy(dst_hbm.at[row], dstb, semi)

      def wait_idx(b):
        srcb, dstb, _, _, semi, _ = bufs[b]
        pltpu.make_async_copy(src_hbm.at[0], srcb, semi).wait()
        pltpu.make_async_copy(dst_hbm.at[0], dstb, semi).wait()

      def fire_gather(b):
        srcb, _, rows, semg, _, _ = bufs[b]
        pltpu.async_copy(t_hbm.at[srcb, pl.ds(coff, TW)], rows, semg)

      def wait_gather(b):
        srcb, _, rows, semg, _, _ = bufs[b]
        pltpu.make_async_copy(t_hbm.at[srcb, pl.ds(coff, TW)], rows,
                              semg).wait()

      def fire_scatter(b):
        _, dstb, rows, _, _, sems = bufs[b]
        pltpu.async_copy(rows, acc_sh.at[dstb], sems, add=True)

      def wait_scatter(b):
        _, dstb, rows, _, _, sems = bufs[b]
        pltpu.make_async_copy(rows, acc_sh.at[dstb], sems).wait()

      def count(b):
        if b == parity:
          dstb = bufs[b][1]
          for g in range(CHUNK // 16):
            dvec = dstb[pl.ds(g * 16, 16)]
            plsc.addupdate_scatter(deg_v, [dvec], ones16)

      fire_idx(0, 0)
      fire_idx(1, 1)
      wait_idx(0)
      fire_gather(0)

      def body(j2, _):
        j = 2 * j2
        wait_gather(0)
        count(0)
        fire_scatter(0)
        wait_idx(1)
        fire_gather(1)
        wait_scatter(0)
        fire_idx(j + 2, 0)
        wait_gather(1)
        count(1)
        fire_scatter(1)
        wait_idx(0)
        fire_gather(0)
        wait_scatter(1)
        fire_idx(j + 3, 1)
        return _

      lax.fori_loop(0, CHUNKS_PER_TILE // 2 - 1, body, 0)
      wait_gather(0)
      count(0)
      fire_scatter(0)
      wait_idx(1)
      fire_gather(1)
      wait_scatter(0)
      wait_gather(1)
      count(1)
      fire_scatter(1)
      wait_scatter(1)
      pltpu.sync_copy(deg_v, parts_hbm.at[prow])
      plsc.subcore_barrier()
      base = s * ROWS_PER_TILE
      pltpu.sync_copy(acc_sh.at[pl.ds(base, ROWS_PER_TILE)],
                      out_hbm.at[pl.ds(base, ROWS_PER_TILE)])

    @pl.when(c == 0)
    def _():
      run_core(h_hbm, 0, out0_hbm, 0, s)

    @pl.when(c == 1)
    def _():
      run_core(h_hbm, TW, out1_hbm, 1, NUM_TILES + s)

  return agg_kernel(h, src_r, dst_r)


def _layernorm(x, g, b, eps=1e-5):
  mu = jnp.mean(x, axis=-1, keepdims=True)
  var = jnp.mean((x - mu) ** 2, axis=-1, keepdims=True)
  return (x - mu) * jax.lax.rsqrt(var + eps) * g + b


ROW_BLK = 2000


def _dense_body(h_ref, a0_ref, a1_ref, deg_ref, A0_ref, A1_ref, bm_ref,
                WsT_ref, bs_ref, W1T_ref, bf1_ref, W2T_ref, bf2_ref,
                g1_ref, b1_ref, g2_ref, b2_ref, out_ref):
  h = h_ref[...]
  deg = jnp.sum(deg_ref[...], axis=1, keepdims=True)
  dot = functools.partial(jnp.dot, preferred_element_type=jnp.float32)
  pre = dot(a0_ref[...], A0_ref[...]) + dot(a1_ref[...], A1_ref[...])
  pre = pre + bm_ref[...] * deg
  agg = pre / jnp.maximum(deg, 1.0)
  x = h + jnp.maximum(dot(h, WsT_ref[...]) + bs_ref[...] + agg, 0.0)
  h1 = _layernorm(x, g1_ref[...], b1_ref[...])
  hid = dot(h1, W1T_ref[...]) + bf1_ref[...]
  hid = hid * 0.5 * (1.0 + lax.erf(hid * (2.0 ** -0.5)))
  ffn = dot(hid, W2T_ref[...]) + bf2_ref[...]
  out_ref[...] = _layernorm(h1 + ffn, g2_ref[...], b2_ref[...])


def _tc_dense(h, acc0, acc1, deg, A0, A1, bm, WsT, bs, W1T, bf1, W2T, bf2,
              g1, b1, g2, b2):
  grid = (N // ROW_BLK,)
  row_spec = lambda w: pl.BlockSpec((ROW_BLK, w), lambda i: (i, 0))
  full = lambda a: pl.BlockSpec(a.shape, lambda i: (0,) * a.ndim)
  return pl.pallas_call(
      _dense_body,
      grid=grid,
      in_specs=[
          row_spec(D), row_spec(TW), row_spec(TW), row_spec(2 * NUM_TILES),
          full(A0), full(A1), full(bm), full(WsT), full(bs),
          full(W1T), full(bf1), full(W2T), full(bf2),
          full(g1), full(b1), full(g2), full(b2),
      ],
      out_specs=row_spec(D),
      out_shape=jax.ShapeDtypeStruct((N, D), jnp.float32),
  )(h, acc0, acc1, deg, A0, A1, bm, WsT, bs, W1T, bf1, W2T, bf2,
    g1, b1, g2, b2)


@jax.jit
def kernel(h, edge_index, Wm, bm, Ws, bs, g1, beta1, W1, bf1, W2, bf2,
           g2, beta2):
  src = jnp.concatenate(
      [edge_index[0].astype(jnp.int32),
       jnp.zeros((EPAD - E,), jnp.int32)]).reshape(EPAD // CHUNK, CHUNK)
  dst = jnp.concatenate(
      [edge_index[1].astype(jnp.int32),
       N + jax.lax.rem(jnp.arange(EPAD - E, dtype=jnp.int32),
                       jnp.int32(NPAD - N))]).reshape(EPAD // CHUNK, CHUNK)

  acc0, acc1, parts = _sc_aggregate(h, src, dst)
  deg = parts.T

  WmT = Wm.T
  A0 = WmT[: D // 2]
  A1 = WmT[D // 2 :]

  return _tc_dense(
      h, acc0, acc1, deg, A0, A1, bm[None, :], Ws.T, bs[None, :],
      W1.T, bf1[None, :], W2.T, bf2[None, :], g1[None, :], beta1[None, :],
      g2[None, :], beta2[None, :])


# speedup vs baseline: 1.8604x; 1.0000x over previous
import functools

import jax
import jax.numpy as jnp
from jax import lax
from jax.experimental import pallas as pl
from jax.experimental.pallas import tpu as pltpu
from jax.experimental.pallas import tpu_sc as plsc

N = 10000
D = 256
E = 160000

NUM_TILES = 16
CHUNK = 112
EPAD = 161280
EDGES_PER_TILE = EPAD // NUM_TILES
CHUNKS_PER_TILE = EDGES_PER_TILE // CHUNK
TW = 128
NPAD = 10240
ROWS_PER_TILE = NPAD // NUM_TILES
ZROWS = 80


def _sc_aggregate(h, src_r, dst_r):
  mesh = plsc.VectorSubcoreMesh(core_axis_name="c", subcore_axis_name="s")

  @functools.partial(
      pl.kernel,
      mesh=mesh,
      compiler_params=pltpu.CompilerParams(needs_layout_passes=False),
      out_type=[
          jax.ShapeDtypeStruct((NPAD, TW), jnp.float32),
          jax.ShapeDtypeStruct((NPAD, TW), jnp.float32),
          jax.ShapeDtypeStruct((2 * NUM_TILES, NPAD), jnp.float32),
      ],
      scratch_types=[
          pltpu.VMEM((CHUNK,), jnp.int32),
          pltpu.VMEM((CHUNK,), jnp.int32),
          pltpu.VMEM((CHUNK,), jnp.int32),
          pltpu.VMEM((CHUNK,), jnp.int32),
          pltpu.VMEM((CHUNK, TW), jnp.float32),
          pltpu.VMEM((CHUNK, TW), jnp.float32),
          pltpu.VMEM((NPAD,), jnp.float32),
          pltpu.VMEM_SHARED((NPAD, TW), jnp.float32),
          pltpu.SemaphoreType.DMA,
          pltpu.SemaphoreType.DMA,
          pltpu.SemaphoreType.DMA,
          pltpu.SemaphoreType.DMA,
          pltpu.SemaphoreType.DMA,
          pltpu.SemaphoreType.DMA,
      ],
  )
  def agg_kernel(h_hbm, src_hbm, dst_hbm,
                 out0_hbm, out1_hbm, parts_hbm,
                 src0_v, dst0_v, src1_v, dst1_v, rows0_v, rows1_v,
                 deg_v, acc_sh, semg0, semg1, semi0, semi1, sems0, sems1):
    c = lax.axis_index("c")
    s = lax.axis_index("s")

    zvec = jnp.zeros((16,), jnp.float32)
    ones16 = jnp.ones((16,), jnp.float32)

    def zrow(r, _):
      for j in range(TW // 16):
        rows0_v[r, pl.ds(j * 16, 16)] = zvec
      return _

    lax.fori_loop(0, ZROWS, zrow, 0)

    def zdeg(r, _):
      deg_v[pl.ds(r * 16, 16)] = zvec
      return _

    lax.fori_loop(0, NPAD // 16, zdeg, 0)

    for z in range(ROWS_PER_TILE // ZROWS):
      pltpu.sync_copy(
          rows0_v.at[pl.ds(0, ZROWS)],
          acc_sh.at[pl.ds(s * ROWS_PER_TILE + z * ZROWS, ZROWS)])

    plsc.subcore_barrier()

    def run_core(t_hbm, coff, out_hbm, parity, prow):
      bufs = ((src0_v, dst0_v, rows0_v, semg0, semi0, sems0),
              (src1_v, dst1_v, rows1_v, semg1, semi1, sems1))

      def fire_idx(j, b):
        srcb, dstb, _, _, semi, _ = bufs[b]
        row = s * CHUNKS_PER_TILE + j
        pltpu.async_copy(src_hbm.at[row], srcb, semi)
        pltpu.async_copy(dst_hbm.at[row], dstb, semi)

      def wait_idx(b):
        srcb, dstb, _, _, semi, _ = bufs[b]
        pltpu.make_async_copy(src_hbm.at[0], srcb, semi).wait()
        pltpu.make_async_copy(dst_hbm.at[0], dstb, semi).wait()

      def fire_gather(b):
        srcb, _, rows, semg, _, _ = bufs[b]
        pltpu.async_copy(t_hbm.at[srcb, pl.ds(coff, TW)], rows, semg)

      def wait_gather(b):
        srcb, _, rows, semg, _, _ = bufs[b]
        pltpu.make_async_copy(t_hbm.at[srcb, pl.ds(coff, TW)], rows,
                              semg).wait()

      def fire_scatter(b):
        _, dstb, rows, _, _, sems = bufs[b]
        pltpu.async_copy(rows, acc_sh.at[dstb], sems, add=True)

      def wait_scatter(b):
        _, dstb, rows, _, _, sems = bufs[b]
        pltpu.make_async_copy(rows, acc_sh.at[dstb], sems).wait()

      def count(b):
        if b == parity:
          dstb = bufs[b][1]
          for g in range(CHUNK // 16):
            dvec = dstb[pl.ds(g * 16, 16)]
            plsc.addupdate_scatter(deg_v, [dvec], ones16)

      fire_idx(0, 0)
      fire_idx(1, 1)
      wait_idx(0)
      fire_gather(0)

      def body(j2, _):
        j = 2 * j2
        wait_gather(0)
        count(0)
        fire_scatter(0)
        wait_idx(1)
        fire_gather(1)
        wait_scatter(0)
        fire_idx(j + 2, 0)
        wait_gather(1)
        count(1)
        fire_scatter(1)
        wait_idx(0)
        fire_gather(0)
        wait_scatter(1)
        fire_idx(j + 3, 1)
        return _

      lax.fori_loop(0, CHUNKS_PER_TILE // 2 - 1, body, 0)
      wait_gather(0)
      count(0)
      fire_scatter(0)
      wait_idx(1)
      fire_gather(1)
      wait_scatter(0)
      wait_gather(1)
      count(1)
      fire_scatter(1)
      wait_scatter(1)
      pltpu.sync_copy(deg_v, parts_hbm.at[prow])
      plsc.subcore_barrier()
      base = s * ROWS_PER_TILE
      pltpu.sync_copy(acc_sh.at[pl.ds(base, ROWS_PER_TILE)],
                      out_hbm.at[pl.ds(base, ROWS_PER_TILE)])

    @pl.when(c == 0)
    def _():
      run_core(h_hbm, 0, out0_hbm, 0, s)

    @pl.when(c == 1)
    def _():
      run_core(h_hbm, TW, out1_hbm, 1, NUM_TILES + s)

  return agg_kernel(h, src_r, dst_r)


def _layernorm(x, g, b, eps=1e-5):
  mu = jnp.mean(x, axis=-1, keepdims=True)
  var = jnp.mean((x - mu) ** 2, axis=-1, keepdims=True)
  return (x - mu) * jax.lax.rsqrt(var + eps) * g + b


ROW_BLK = 2000


def _dense_body(h_ref, a0_ref, a1_ref, deg_ref, A0_ref, A1_ref, bm_ref,
                WsT_ref, bs_ref, W1T_ref, bf1_ref, W2T_ref, bf2_ref,
                g1_ref, b1_ref, g2_ref, b2_ref, out_ref):
  h = h_ref[...]
  deg = jnp.sum(deg_ref[...], axis=1, keepdims=True)
  bf = jnp.bfloat16

  def dot(x, w_ref):
    return jnp.dot(x.astype(bf), w_ref[...],
                   preferred_element_type=jnp.float32)

  pre = dot(a0_ref[...], A0_ref) + dot(a1_ref[...], A1_ref)
  pre = pre + bm_ref[...] * deg
  agg = pre / jnp.maximum(deg, 1.0)
  x = h + jnp.maximum(dot(h, WsT_ref) + bs_ref[...] + agg, 0.0)
  h1 = _layernorm(x, g1_ref[...], b1_ref[...])
  hid = dot(h1, W1T_ref) + bf1_ref[...]
  hid = hid * 0.5 * (1.0 + lax.erf(hid * (2.0 ** -0.5)))
  ffn = dot(hid, W2T_ref) + bf2_ref[...]
  out_ref[...] = _layernorm(h1 + ffn, g2_ref[...], b2_ref[...])


def _tc_dense(h, acc0, acc1, deg, A0, A1, bm, WsT, bs, W1T, bf1, W2T, bf2,
              g1, b1, g2, b2):
  grid = (N // ROW_BLK,)
  row_spec = lambda w: pl.BlockSpec((ROW_BLK, w), lambda i: (i, 0))
  full = lambda a: pl.BlockSpec(a.shape, lambda i: (0,) * a.ndim)
  return pl.pallas_call(
      _dense_body,
      grid=grid,
      in_specs=[
          row_spec(D), row_spec(TW), row_spec(TW), row_spec(2 * NUM_TILES),
          full(A0), full(A1), full(bm), full(WsT), full(bs),
          full(W1T), full(bf1), full(W2T), full(bf2),
          full(g1), full(b1), full(g2), full(b2),
      ],
      out_specs=row_spec(D),
      out_shape=jax.ShapeDtypeStruct((N, D), jnp.float32),
  )(h, acc0, acc1, deg, A0, A1, bm, WsT, bs, W1T, bf1, W2T, bf2,
    g1, b1, g2, b2)


@jax.jit
def kernel(h, edge_index, Wm, bm, Ws, bs, g1, beta1, W1, bf1, W2, bf2,
           g2, beta2):
  src = jnp.concatenate(
      [edge_index[0].astype(jnp.int32),
       jnp.zeros((EPAD - E,), jnp.int32)]).reshape(EPAD // CHUNK, CHUNK)
  dst = jnp.concatenate(
      [edge_index[1].astype(jnp.int32),
       N + jax.lax.rem(jnp.arange(EPAD - E, dtype=jnp.int32),
                       jnp.int32(NPAD - N))]).reshape(EPAD // CHUNK, CHUNK)

  acc0, acc1, parts = _sc_aggregate(h, src, dst)
  deg = parts.T

  WmT = Wm.T
  A0 = WmT[: D // 2]
  A1 = WmT[D // 2 :]

  bf = jnp.bfloat16
  return _tc_dense(
      h, acc0, acc1, deg, A0.astype(bf), A1.astype(bf), bm[None, :],
      Ws.T.astype(bf), bs[None, :], W1.T.astype(bf), bf1[None, :],
      W2.T.astype(bf), bf2[None, :], g1[None, :], beta1[None, :],
      g2[None, :], beta2[None, :])
